# Initial kernel scaffold; baseline (speedup 1.0000x reference)
import jax
import jax.numpy as jnp
from jax.experimental import pallas as pl


def kernel(hidden, velocity, mu_prev, ln1_w, ln2_w, Wq, Wk, Wv, Wo, Wmq, Wmk, Wmv, qn_w, kn_w, mu, Wmp, Wci, bci, Wco, bco, Wr, Wg, Wu, Wd, positions, token_ids):
    raise NotImplementedError("write your pallas kernel here")


# speedup vs baseline: 2.0806x; 2.0806x over previous
import jax
import jax.numpy as jnp
from jax.experimental import pallas as pl
from jax.experimental.pallas import tpu as pltpu

H = 16; DH = 64; KV = 4; D = 1024; CH = 64; I = 2816; E = 8
DT = 0.1; EPS = 1e-06; THETA = 10000.0
G = H // KV

TOK_T = 256
MOE_T = 256
NT = 16
I_BLK = 1408
IT = I // I_BLK


def _rms(x, eps=EPS):
    return jax.lax.rsqrt(jnp.mean(x * x, axis=-1, keepdims=True) + eps)


def _mm_t(a, w):
    return jax.lax.dot_general(a, w, (((1,), (1,)), ((), ())),
                               preferred_element_type=jnp.float32)


def _qkv_kernel(hid_ref, mu_ref, ln1_ref, wh_ref, wm_ref, qn_ref, kn_ref,
                cos_ref, sin_ref, q_ref, k_ref, v_ref):
    hid = hid_ref[...]
    h = hid * _rms(hid) * ln1_ref[...]
    qkv = _mm_t(h, wh_ref[...]) + _mm_t(mu_ref[...], wm_ref[...])
    cos = cos_ref[...]
    sin = sin_ref[...]

    def norm_rope(x, w):
        x = x * _rms(x) * w
        half = jnp.concatenate([-x[:, DH // 2:], x[:, :DH // 2]], axis=-1)
        return x * cos + half * sin

    q_parts = [norm_rope(qkv[:, i * DH:(i + 1) * DH], qn_ref[...])
               for i in range(H)]
    k_parts = [norm_rope(qkv[:, H * DH + i * DH:H * DH + (i + 1) * DH],
                         kn_ref[...]) for i in range(KV)]
    q_ref[...] = jnp.concatenate(q_parts, axis=-1)
    k_ref[...] = jnp.concatenate(k_parts, axis=-1)
    v_ref[...] = qkv[:, (H + KV) * DH:]


def _attn_kernel(q_ref, k_ref, v_ref, o_ref):
    q = q_ref[...]
    k = k_ref[...]
    v = v_ref[...]
    scale = 1.0 / jnp.sqrt(jnp.float32(DH))
    outs = []
    for h in range(H):
        qh = q[:, h * DH:(h + 1) * DH]
        kv = h // G
        kh = k[:, kv * DH:(kv + 1) * DH]
        vh = v[:, kv * DH:(kv + 1) * DH]
        s = jax.lax.dot_general(qh, kh, (((1,), (1,)), ((), ())),
                                preferred_element_type=jnp.float32) * scale
        m = jnp.max(s, axis=-1, keepdims=True)
        p = jnp.exp(s - m)
        p = p / jnp.sum(p, axis=-1, keepdims=True)
        outs.append(jnp.dot(p, vh, preferred_element_type=jnp.float32))
    o_ref[...] = jnp.concatenate(outs, axis=-1)


def _post_kernel(ao_ref, vel_ref, hid_ref, tid_ref, mu_ref, ln2_ref,
                 wo_ref, wmp_ref, wcia_ref, wcib_ref, bci_ref, wco_ref,
                 bco_ref, wr_ref, h2_ref, vn_ref, mc_ref, hn_ref,
                 eid_ref, rnk_ref, cnt_ref, carry_ref):
    i = pl.program_id(0)

    @pl.when(i == 0)
    def _():
        carry_ref[...] = jnp.zeros_like(carry_ref)

    out = ao_ref[...]
    vel = vel_ref[...]
    ao = _mm_t(out, wo_ref[...])
    mu_ctx = mu_ref[...] + _mm_t(ao, wmp_ref[...])
    ctrl = _mm_t(ao, wcia_ref[...]) + _mm_t(vel, wcib_ref[...]) + bci_ref[...]
    ctrl = ctrl * jax.nn.sigmoid(ctrl)
    co = _mm_t(ctrl, wco_ref[...]) + bco_ref[...]
    alpha = jax.nn.sigmoid(co[:, :D])
    sp_in = co[:, D:2 * D]
    beta = jnp.minimum(
        jnp.maximum(sp_in, 0.0) + jnp.log1p(jnp.exp(-jnp.abs(sp_in))), 2.0)
    gate = jax.nn.sigmoid(co[:, 2 * D:])
    err = ao - mu_ctx
    v_next = jnp.clip(alpha * vel - beta * err, -10.0, 10.0)
    h2 = hid_ref[...] + ao + DT * gate * v_next
    hn = h2 * _rms(h2) * ln2_ref[...]

    h2_ref[...] = h2
    vn_ref[...] = v_next
    mc_ref[...] = mu_ctx
    hn_ref[...] = hn

    rT = jax.lax.dot_general(wr_ref[...], mu_ctx, (((1,), (1,)), ((), ())),
                             preferred_element_type=jnp.float32)
    tid = tid_ref[0]
    base = jax.lax.rem(tid, E)
    erow = jax.lax.broadcasted_iota(jnp.int32, (E, TOK_T), 0)
    comb = jnp.where(base == erow, 10.0, 0.0) + rT
    mx = jnp.max(comb, axis=0, keepdims=True)
    eid = jnp.min(jnp.where(comb >= mx, erow, E), axis=0, keepdims=True)

    oh = (erow == eid).astype(jnp.float32)
    lt = jax.lax.broadcasted_iota(jnp.int32, (TOK_T, TOK_T), 0)
    ut = (lt <= jax.lax.broadcasted_iota(jnp.int32, (TOK_T, TOK_T), 1))
    cum = jnp.dot(oh, ut.astype(jnp.float32),
                  preferred_element_type=jnp.float32)
    carry = carry_ref[:, :1].astype(jnp.float32)
    rank = jnp.sum(oh * (cum - 1.0 + carry), axis=0, keepdims=True)
    cnt = jnp.sum(oh, axis=1, keepdims=True)
    carry_ref[...] = carry_ref[...] + jnp.broadcast_to(
        cnt.astype(jnp.int32), carry_ref.shape)
    cnt_ref[...] = carry_ref[...]

    eid_ref[...] = eid.astype(jnp.int32).reshape(1, 1, TOK_T)
    rnk_ref[...] = rank.astype(jnp.int32).reshape(1, 1, TOK_T)


def _moe_kernel(te_ref, x_ref, wg_ref, wu_ref, wd_ref, y_ref):
    i = pl.program_id(1)
    x = x_ref[...]
    g = jax.lax.dot_general(x, wg_ref[0], (((1,), (1,)), ((), ())),
                            preferred_element_type=jnp.float32)
    u = jax.lax.dot_general(x, wu_ref[0], (((1,), (1,)), ((), ())),
                            preferred_element_type=jnp.float32)
    a = g * jax.nn.sigmoid(g) * u
    part = jax.lax.dot_general(a, wd_ref[0], (((1,), (1,)), ((), ())),
                               preferred_element_type=jnp.float32)

    @pl.when(i == 0)
    def _():
        y_ref[...] = jnp.zeros_like(y_ref)

    y_ref[...] += part


@jax.jit
def kernel(hidden, velocity, mu_prev, ln1_w, ln2_w, Wq, Wk, Wv, Wo, Wmq, Wmk,
           Wmv, qn_w, kn_w, mu, Wmp, Wci, bci, Wco, bco, Wr, Wg, Wu, Wd,
           positions, token_ids):
    n = hidden.shape[0]
    nt = n // TOK_T

    inv_freq = 1.0 / (THETA ** (jnp.arange(0, DH, 2, dtype=jnp.float32) / DH))
    freqs = positions.astype(jnp.float32)[:, None] * inv_freq[None, :]
    emb = jnp.concatenate([freqs, freqs], axis=-1)
    cos = jnp.cos(emb)
    sin = jnp.sin(emb)

    wh = jnp.concatenate([Wq, Wk, Wv], axis=0)
    wm = jnp.concatenate([Wmq, Wmk, Wmv], axis=0)

    tok_spec = pl.BlockSpec((TOK_T, D), lambda i: (i, 0))

    def full(*shape):
        return pl.BlockSpec(shape, lambda i, _s=len(shape): (0,) * _s)

    q, k, v = pl.pallas_call(
        _qkv_kernel,
        grid=(nt,),
        in_specs=[tok_spec, tok_spec, full(1, D),
                  full((H + 2 * KV) * DH, D), full((H + 2 * KV) * DH, D),
                  full(1, DH), full(1, DH),
                  pl.BlockSpec((TOK_T, DH), lambda i: (i, 0)),
                  pl.BlockSpec((TOK_T, DH), lambda i: (i, 0))],
        out_specs=[tok_spec,
                   pl.BlockSpec((TOK_T, KV * DH), lambda i: (i, 0)),
                   pl.BlockSpec((TOK_T, KV * DH), lambda i: (i, 0))],
        out_shape=[jax.ShapeDtypeStruct((n, D), jnp.float32),
                   jax.ShapeDtypeStruct((n, KV * DH), jnp.float32),
                   jax.ShapeDtypeStruct((n, KV * DH), jnp.float32)],
    )(hidden, mu_prev, ln1_w.reshape(1, D), wh, wm,
      qn_w.reshape(1, DH), kn_w.reshape(1, DH), cos, sin)

    attn = pl.pallas_call(
        _attn_kernel,
        grid=(nt,),
        in_specs=[tok_spec,
                  pl.BlockSpec((n, KV * DH), lambda i: (0, 0)),
                  pl.BlockSpec((n, KV * DH), lambda i: (0, 0))],
        out_specs=tok_spec,
        out_shape=jax.ShapeDtypeStruct((n, D), jnp.float32),
    )(q, k, v)

    tid3 = token_ids.reshape(nt, 1, TOK_T)
    h2, v_next, mu_ctx, hn, eid3, rnk3, cnts = pl.pallas_call(
        _post_kernel,
        grid=(nt,),
        in_specs=[tok_spec, tok_spec, tok_spec,
                  pl.BlockSpec((1, 1, TOK_T), lambda i: (i, 0, 0)),
                  full(1, D), full(1, D),
                  full(D, D), full(D, D),
                  full(CH, D), full(CH, D), full(1, CH),
                  full(3 * D, CH), full(1, 3 * D), full(E, D)],
        out_specs=[tok_spec, tok_spec, tok_spec, tok_spec,
                   pl.BlockSpec((1, 1, TOK_T), lambda i: (i, 0, 0)),
                   pl.BlockSpec((1, 1, TOK_T), lambda i: (i, 0, 0)),
                   pl.BlockSpec((E, 128), lambda i: (0, 0))],
        out_shape=[jax.ShapeDtypeStruct((n, D), jnp.float32),
                   jax.ShapeDtypeStruct((n, D), jnp.float32),
                   jax.ShapeDtypeStruct((n, D), jnp.float32),
                   jax.ShapeDtypeStruct((n, D), jnp.float32),
                   jax.ShapeDtypeStruct((nt, 1, TOK_T), jnp.int32),
                   jax.ShapeDtypeStruct((nt, 1, TOK_T), jnp.int32),
                   jax.ShapeDtypeStruct((E, 128), jnp.int32)],
        scratch_shapes=[pltpu.VMEM((E, 128), jnp.int32)],
    )(attn, velocity, hidden, tid3, mu.reshape(1, D), ln2_w.reshape(1, D),
      Wo, Wmp, Wci[:, :D], Wci[:, D:], bci.reshape(1, CH), Wco,
      bco.reshape(1, 3 * D), Wr)

    eid = eid3.reshape(n)
    rank = rnk3.reshape(n)
    counts = cnts[:, 0]

    ntiles_e = (counts + MOE_T - 1) // MOE_T
    cum_tiles = jnp.cumsum(ntiles_e)
    tile_start = cum_tiles - ntiles_e
    jrange = jnp.arange(NT, dtype=jnp.int32)
    tile_expert = jnp.minimum(
        jnp.sum((jrange[:, None] >= cum_tiles[None, :]).astype(jnp.int32),
                axis=1), E - 1).astype(jnp.int32)
    pos = tile_start[eid] * MOE_T + rank

    x = jnp.zeros((NT * MOE_T, D), jnp.float32).at[pos].set(hn)

    y = pl.pallas_call(
        _moe_kernel,
        grid_spec=pltpu.PrefetchScalarGridSpec(
            num_scalar_prefetch=1,
            grid=(NT, IT),
            in_specs=[pl.BlockSpec((MOE_T, D), lambda j, i, te: (j, 0)),
                      pl.BlockSpec((1, I_BLK, D),
                                   lambda j, i, te: (te[j], i, 0)),
                      pl.BlockSpec((1, I_BLK, D),
                                   lambda j, i, te: (te[j], i, 0)),
                      pl.BlockSpec((1, D, I_BLK),
                                   lambda j, i, te: (te[j], 0, i))],
            out_specs=pl.BlockSpec((MOE_T, D), lambda j, i, te: (j, 0)),
        ),
        out_shape=jax.ShapeDtypeStruct((NT * MOE_T, D), jnp.float32),
    )(tile_expert, x, Wg, Wu, Wd)

    hidden_next = h2 + y[pos]
    return hidden_next, v_next, mu_ctx
